# Initial kernel scaffold; baseline (speedup 1.0000x reference)
import jax
import jax.numpy as jnp
from jax.experimental import pallas as pl


def kernel(cards, rank_emb, suit_emb):
    raise NotImplementedError("write your pallas kernel here")


# speedup vs baseline: 8.0649x; 8.0649x over previous
import functools

import jax
import jax.numpy as jnp
from jax import lax
from jax.experimental import pallas as pl
from jax.experimental.pallas import tpu as pltpu
from jax.experimental.pallas import tpu_sc as plsc

L = 16
NC = 2
NS = 16
NW = NC * NS

D = 12


@functools.lru_cache(maxsize=None)
def _make_lookup(n_cards: int, chunk: int):
    assert n_cards % (NW * chunk * 2) == 0
    per_tile = n_cards // NW
    n_chunks = per_tile // chunk
    groups = chunk // L

    mesh = plsc.VectorSubcoreMesh(core_axis_name="c", subcore_axis_name="s")

    @functools.partial(
        pl.kernel,
        out_type=jax.ShapeDtypeStruct((n_cards * D,), jnp.float32),
        mesh=mesh,
        compiler_params=pltpu.CompilerParams(needs_layout_passes=False),
        scratch_types=[
            pltpu.VMEM((48,), jnp.float32),
            pltpu.VMEM((16,), jnp.float32),
            pltpu.VMEM((24 * D,), jnp.float32),
            pltpu.VMEM((chunk,), jnp.int32),
            pltpu.VMEM((chunk * D,), jnp.float32),
            pltpu.VMEM((chunk * D,), jnp.float32),
            pltpu.SemaphoreType.DMA,
        ],
    )
    def lookup(cards_hbm, rank_hbm, suit_hbm, out_hbm,
               rank_v, suit_v, table_v, cards_v, out_v0, out_v1, sem):
        out_bufs = (out_v0, out_v1)
        wid = lax.axis_index("s") * NC + lax.axis_index("c")
        pltpu.sync_copy(rank_hbm, rank_v)
        pltpu.sync_copy(suit_hbm, suit_v)

        lane = lax.iota(jnp.int32, L)
        for k in range(24 * D // L):
            pos = k * L + lane
            c = pos // D
            j = pos - c * D
            is_rank = j < 8
            ridx = jnp.where(is_rank, (c // 4) * 8 + j, 0)
            sidx = jnp.where(is_rank, 0, (c - (c // 4) * 4) * 4 + (j - 8))
            rv = plsc.load_gather(rank_v, [ridx])
            sv = plsc.load_gather(suit_v, [sidx])
            table_v[pl.ds(k * L, L)] = jnp.where(is_rank, rv, sv)

        lane12 = lane * D
        base0 = wid * per_tile

        def do_chunk(ci, obuf):
            cbase = base0 + ci * chunk
            pltpu.sync_copy(cards_hbm.at[pl.ds(cbase, chunk)], cards_v)

            def grp(g, _):
                c16 = cards_v[pl.ds(g * L, L)]
                tb = c16 * D
                op = g * (L * D) + lane12
                for j in range(D):
                    v = plsc.load_gather(table_v, [tb + j])
                    plsc.store_scatter(obuf, [op + j], v)
                return 0

            lax.fori_loop(0, groups, grp, 0)
            pltpu.async_copy(obuf, out_hbm.at[pl.ds(cbase * D, chunk * D)], sem)

        def body(pi, _):
            for b in range(2):
                ci = pi * 2 + b

                @pl.when(pi >= 1)
                def _():
                    pltpu.make_async_copy(
                        out_bufs[b],
                        out_hbm.at[pl.ds(0, chunk * D)],
                        sem,
                    ).wait()

                do_chunk(ci, out_bufs[b])
            return 0

        lax.fori_loop(0, n_chunks // 2, body, 0)
        for b in range(2):
            pltpu.make_async_copy(
                out_bufs[b],
                out_hbm.at[pl.ds(0, chunk * D)],
                sem,
            ).wait()

    return lookup


def kernel(cards, rank_emb, suit_emb):
    b, s = cards.shape
    n = b * s
    cards_flat = cards.reshape(-1).astype(jnp.int32)
    out = _make_lookup(n, 3200)(
        cards_flat, rank_emb.reshape(-1), suit_emb.reshape(-1)
    )
    return out.reshape(b, s, D)


# speedup vs baseline: 9.0262x; 1.1192x over previous
import functools

import jax
import jax.numpy as jnp
from jax import lax
from jax.experimental import pallas as pl
from jax.experimental.pallas import tpu as pltpu
from jax.experimental.pallas import tpu_sc as plsc

L = 16
NC = 2
NS = 16
NW = NC * NS

D = 12


@functools.lru_cache(maxsize=None)
def _make_lookup(n_cards: int, chunk: int):
    assert n_cards % (NW * chunk * 2) == 0
    per_tile = n_cards // NW
    n_chunks = per_tile // chunk
    groups = chunk // L

    mesh = plsc.VectorSubcoreMesh(core_axis_name="c", subcore_axis_name="s")

    @functools.partial(
        pl.kernel,
        out_type=jax.ShapeDtypeStruct((n_cards * D,), jnp.float32),
        mesh=mesh,
        compiler_params=pltpu.CompilerParams(needs_layout_passes=False),
        scratch_types=[
            pltpu.VMEM((48,), jnp.float32),
            pltpu.VMEM((16,), jnp.float32),
            pltpu.VMEM((24 * D,), jnp.float32),
            pltpu.VMEM((chunk,), jnp.int32),
            pltpu.VMEM((chunk * D,), jnp.float32),
            pltpu.VMEM((chunk * D,), jnp.float32),
            pltpu.SemaphoreType.DMA,
        ],
    )
    def lookup(cards_hbm, rank_hbm, suit_hbm, out_hbm,
               rank_v, suit_v, table_v, cards_v, out_v0, out_v1, sem):
        out_bufs = (out_v0, out_v1)
        wid = lax.axis_index("s") * NC + lax.axis_index("c")
        pltpu.sync_copy(rank_hbm, rank_v)
        pltpu.sync_copy(suit_hbm, suit_v)

        lane = lax.iota(jnp.int32, L)
        for k in range(24 * D // L):
            pos = k * L + lane
            c = pos // D
            j = pos - c * D
            is_rank = j < 8
            ridx = jnp.where(is_rank, (c // 4) * 8 + j, 0)
            sidx = jnp.where(is_rank, 0, (c - (c // 4) * 4) * 4 + (j - 8))
            rv = plsc.load_gather(rank_v, [ridx])
            sv = plsc.load_gather(suit_v, [sidx])
            table_v[pl.ds(k * L, L)] = jnp.where(is_rank, rv, sv)

        lane12 = lane * D
        base0 = wid * per_tile

        def do_chunk(ci, obuf):
            cbase = base0 + ci * chunk
            pltpu.sync_copy(cards_hbm.at[pl.ds(cbase, chunk)], cards_v)

            @plsc.parallel_loop(0, groups, unroll=4)
            def _(g):
                c16 = cards_v[pl.ds(g * L, L)]
                tb = c16 * D
                op = g * (L * D) + lane12
                for j in range(D):
                    v = plsc.load_gather(table_v, [tb + j])
                    plsc.store_scatter(obuf, [op + j], v)
            pltpu.async_copy(obuf, out_hbm.at[pl.ds(cbase * D, chunk * D)], sem)

        def body(pi, _):
            for b in range(2):
                ci = pi * 2 + b

                @pl.when(pi >= 1)
                def _():
                    pltpu.make_async_copy(
                        out_bufs[b],
                        out_hbm.at[pl.ds(0, chunk * D)],
                        sem,
                    ).wait()

                do_chunk(ci, out_bufs[b])
            return 0

        lax.fori_loop(0, n_chunks // 2, body, 0)
        for b in range(2):
            pltpu.make_async_copy(
                out_bufs[b],
                out_hbm.at[pl.ds(0, chunk * D)],
                sem,
            ).wait()

    return lookup


def kernel(cards, rank_emb, suit_emb):
    b, s = cards.shape
    n = b * s
    cards_flat = cards.reshape(-1).astype(jnp.int32)
    out = _make_lookup(n, 3200)(
        cards_flat, rank_emb.reshape(-1), suit_emb.reshape(-1)
    )
    return out.reshape(b, s, D)


# speedup vs baseline: 169.9732x; 18.8310x over previous
import functools

import jax
import jax.numpy as jnp
from jax import lax
from jax.experimental import pallas as pl
from jax.experimental.pallas import tpu as pltpu
from jax.experimental.pallas import tpu_sc as plsc

L = 16
NC = 2
NS = 16
NW = NC * NS

D = 12


@functools.lru_cache(maxsize=None)
def _make_lookup(rows: int, cols: int):
    assert rows % 8 == 0 and cols % (128 * NW) == 0
    n_jt = rows // 8
    cw = cols // NW

    mesh = plsc.VectorSubcoreMesh(core_axis_name="c", subcore_axis_name="s")

    @functools.partial(
        pl.kernel,
        out_type=jax.ShapeDtypeStruct((D * rows, cols), jnp.float32),
        mesh=mesh,
        compiler_params=pltpu.CompilerParams(
            needs_layout_passes=False, use_tc_tiling_on_sc=True
        ),
        scratch_types=[
            pltpu.VMEM((48,), jnp.float32),
            pltpu.VMEM((16,), jnp.float32),
            pltpu.VMEM((24 * D,), jnp.float32),
            pltpu.VMEM((8, 512), jnp.int32),
            pltpu.VMEM((8, 512), jnp.int32),
            pltpu.VMEM((D * 8, 512), jnp.float32),
            pltpu.VMEM((D * 8, 512), jnp.float32),
            pltpu.SemaphoreType.DMA,
            pltpu.SemaphoreType.DMA,
        ],
    )
    def lookup(cards_hbm, rank_hbm, suit_hbm, out_hbm,
               rank_v, suit_v, table_v, cv0, cv1, ov0, ov1, csem, osem):
        cbufs = (cv0, cv1)
        obufs = (ov0, ov1)
        wid = lax.axis_index("s") * NC + lax.axis_index("c")
        pltpu.sync_copy(rank_hbm, rank_v)
        pltpu.sync_copy(suit_hbm, suit_v)

        lane = lax.iota(jnp.int32, L)
        for k in range(24 * D // L):
            pos = k * L + lane
            c = pos // D
            j = pos - c * D
            is_rank = j < 8
            ridx = jnp.where(is_rank, (c // 4) * 8 + j, 0)
            sidx = jnp.where(is_rank, 0, (c - (c // 4) * 4) * 4 + (j - 8))
            rv = plsc.load_gather(rank_v, [ridx])
            sv = plsc.load_gather(suit_v, [sidx])
            table_v[pl.ds(k * L, L)] = jnp.where(is_rank, rv, sv)

        ib0 = wid * cw

        def compute(b):
            cbuf = cbufs[b]
            obuf = obufs[b]
            for j in range(8):
                @plsc.parallel_loop(0, cw // L, unroll=4)
                def _(g):
                    c16 = cbuf[j, pl.ds(g * L, L)]
                    tb = c16 * D
                    for k in range(D):
                        v = plsc.load_gather(table_v, [tb + k])
                        obuf[k * 8 + j, pl.ds(g * L, L)] = v

        def start_in(jt, b):
            pltpu.async_copy(
                cards_hbm.at[pl.ds(jt * 8, 8), pl.ds(ib0, cw)],
                cbufs[b], csem,
            )

        def wait_in(b):
            pltpu.make_async_copy(
                cards_hbm.at[pl.ds(0, 8), pl.ds(ib0, cw)],
                cbufs[b], csem,
            ).wait()

        def start_out(jt, b):
            for k in range(D):
                pltpu.async_copy(
                    obufs[b].at[pl.ds(k * 8, 8)],
                    out_hbm.at[pl.ds(k * rows + jt * 8, 8), pl.ds(ib0, cw)],
                    osem,
                )

        def wait_out(b):
            for k in range(D):
                pltpu.make_async_copy(
                    obufs[b].at[pl.ds(k * 8, 8)],
                    out_hbm.at[pl.ds(k * rows, 8), pl.ds(ib0, cw)],
                    osem,
                ).wait()

        start_in(0, 0)

        def body(pi, _):
            for b in range(2):
                jt = pi * 2 + b

                @pl.when(jt < n_jt)
                def _():
                    wait_in(b)

                    @pl.when(jt + 1 < n_jt)
                    def _():
                        start_in(jt + 1, 1 - b)

                    @pl.when(jt >= 2)
                    def _():
                        wait_out(b)

                    compute(b)
                    start_out(jt, b)
            return 0

        lax.fori_loop(0, (n_jt + 1) // 2, body, 0)
        wait_out((n_jt - 2) % 2)
        wait_out((n_jt - 1) % 2)

    return lookup


def kernel(cards, rank_emb, suit_emb):
    b, s = cards.shape
    cards_t = cards.T.astype(jnp.int32)
    out2d = _make_lookup(s, b)(
        cards_t, rank_emb.reshape(-1), suit_emb.reshape(-1)
    )
    return out2d.reshape(D, s, b).transpose(2, 1, 0)


# speedup vs baseline: 217.4189x; 1.2791x over previous
import functools

import jax
import jax.numpy as jnp
from jax import lax
from jax.experimental import pallas as pl
from jax.experimental.pallas import tpu as pltpu
from jax.experimental.pallas import tpu_sc as plsc

L = 16
NC = 2
NS = 16
NW = NC * NS

D = 12


@functools.lru_cache(maxsize=None)
def _make_lookup(rows: int, cols: int):
    assert rows % 8 == 0 and cols % (128 * NW) == 0
    n_jt = rows // 8
    cw = cols // NW

    mesh = plsc.VectorSubcoreMesh(core_axis_name="c", subcore_axis_name="s")

    @functools.partial(
        pl.kernel,
        out_type=jax.ShapeDtypeStruct((D * rows, cols), jnp.float32),
        mesh=mesh,
        compiler_params=pltpu.CompilerParams(
            needs_layout_passes=False, use_tc_tiling_on_sc=True
        ),
        scratch_types=[
            pltpu.VMEM((48,), jnp.float32),
            pltpu.VMEM((16,), jnp.float32),
            pltpu.VMEM((24 * D,), jnp.float32),
            pltpu.VMEM((8, 512), jnp.int32),
            pltpu.VMEM((8, 512), jnp.int32),
            pltpu.VMEM((D * 8, 512), jnp.float32),
            pltpu.VMEM((D * 8, 512), jnp.float32),
            pltpu.SemaphoreType.DMA,
            pltpu.SemaphoreType.DMA,
        ],
    )
    def lookup(cards_hbm, rank_hbm, suit_hbm, out_hbm,
               rank_v, suit_v, table_v, cv0, cv1, ov0, ov1, csem, osem):
        cbufs = (cv0, cv1)
        obufs = (ov0, ov1)
        wid = lax.axis_index("s") * NC + lax.axis_index("c")
        pltpu.sync_copy(rank_hbm, rank_v)
        pltpu.sync_copy(suit_hbm, suit_v)

        lane = lax.iota(jnp.int32, L)
        for v in range(24 * D // L):
            pos = v * L + lane
            k = pos // 24
            c = pos - k * 24
            is_rank = k < 8
            ridx = jnp.where(is_rank, (c // 4) * 8 + k, 0)
            sidx = jnp.where(is_rank, 0, (c - (c // 4) * 4) * 4 + (k - 8))
            rv = plsc.load_gather(rank_v, [ridx])
            sv = plsc.load_gather(suit_v, [sidx])
            table_v[pl.ds(v * L, L)] = jnp.where(is_rank, rv, sv)

        ib0 = wid * cw

        def compute(b):
            cbuf = cbufs[b]
            obuf = obufs[b]
            for j in range(8):
                @plsc.parallel_loop(0, cw // L, unroll=8)
                def _(g):
                    c16 = cbuf[j, pl.ds(g * L, L)]
                    for k in range(D):
                        v = plsc.load_gather(
                            table_v.at[pl.ds(k * 24, 24)], [c16])
                        obuf[k * 8 + j, pl.ds(g * L, L)] = v

        def start_in(jt, b):
            pltpu.async_copy(
                cards_hbm.at[pl.ds(jt * 8, 8), pl.ds(ib0, cw)],
                cbufs[b], csem,
            )

        def wait_in(b):
            pltpu.make_async_copy(
                cards_hbm.at[pl.ds(0, 8), pl.ds(ib0, cw)],
                cbufs[b], csem,
            ).wait()

        def start_out(jt, b):
            for k in range(D):
                pltpu.async_copy(
                    obufs[b].at[pl.ds(k * 8, 8)],
                    out_hbm.at[pl.ds(k * rows + jt * 8, 8), pl.ds(ib0, cw)],
                    osem,
                )

        def wait_out(b):
            for k in range(D):
                pltpu.make_async_copy(
                    obufs[b].at[pl.ds(k * 8, 8)],
                    out_hbm.at[pl.ds(k * rows, 8), pl.ds(ib0, cw)],
                    osem,
                ).wait()

        start_in(0, 0)

        def body(pi, _):
            for b in range(2):
                jt = pi * 2 + b

                @pl.when(jt < n_jt)
                def _():
                    wait_in(b)

                    @pl.when(jt + 1 < n_jt)
                    def _():
                        start_in(jt + 1, 1 - b)

                    @pl.when(jt >= 2)
                    def _():
                        wait_out(b)

                    compute(b)
                    start_out(jt, b)
            return 0

        lax.fori_loop(0, (n_jt + 1) // 2, body, 0)
        wait_out((n_jt - 2) % 2)
        wait_out((n_jt - 1) % 2)

    return lookup


def kernel(cards, rank_emb, suit_emb):
    b, s = cards.shape
    cards_t = cards.T.astype(jnp.int32)
    out2d = _make_lookup(s, b)(
        cards_t, rank_emb.reshape(-1), suit_emb.reshape(-1)
    )
    return out2d.reshape(D, s, b).transpose(2, 1, 0)
